# baseline (speedup 1.0000x reference)
import functools

import jax
import jax.numpy as jnp
from jax import lax
from jax.experimental import pallas as pl
from jax.experimental.pallas import tpu as pltpu
from jax.experimental.pallas import tpu_sc as plsc

N_NODES = 10000
N_EDGES = 320000
N_PAD = 10240
ROWS_PER_TILE = 640
CHUNK = 128
N_SUBCORES = 16

N_CHUNKS_SC = 2560
N_CHUNKS_BUF = 2624
E_SCAT = N_CHUNKS_SC * CHUNK
E_BUF = N_CHUNKS_BUF * CHUNK
K_FULL = N_CHUNKS_SC // N_SUBCORES
K_HALF = N_CHUNKS_SC // (2 * N_SUBCORES)
TRASH = N_NODES


def _issue_idx(src_hbm, src_off, dst_hbm, dst_off, sidx_b, didx_b, sem):
    pltpu.async_copy(src_hbm.at[pl.ds(src_off, CHUNK)], sidx_b, sem)
    pltpu.async_copy(dst_hbm.at[pl.ds(dst_off, CHUNK)], didx_b, sem)


def _wait_idx(src_hbm, dst_hbm, sidx_b, didx_b, sem):
    pltpu.make_async_copy(src_hbm.at[pl.ds(0, CHUNK)], sidx_b, sem).wait()
    pltpu.make_async_copy(dst_hbm.at[pl.ds(0, CHUNK)], didx_b, sem).wait()


def _make_sc_agg_body(edge_split):
    def body(table, srcv, dstv, zblk, agg_out,
             sidx, didx, rows, agg_sh, sa0, sa1, sg0, sg1, *_):
        c = lax.axis_index("c")
        s = lax.axis_index("s")

        pltpu.sync_copy(zblk, agg_sh.at[pl.ds(s * ROWS_PER_TILE, ROWS_PER_TILE)])
        plsc.subcore_barrier()

        if edge_split:
            k_count = K_HALF
            chunk0 = c * (N_CHUNKS_SC // 2) + s
            src_base = 0
        else:
            k_count = K_FULL
            chunk0 = s
            src_base = c * E_BUF

        sems_a = (sa0, sa1)
        sems_g = (sg0, sg1)

        def issue_a(k, b):
            off = (chunk0 + k * N_SUBCORES) * CHUNK
            _issue_idx(srcv, src_base + off, dstv, off,
                       sidx.at[b], didx.at[b], sems_a[b])

        def wait_a(b):
            _wait_idx(srcv, dstv, sidx.at[b], didx.at[b], sems_a[b])

        def issue_g(b):
            pltpu.async_copy(table.at[sidx.at[b]], rows.at[b], sems_g[b])

        def wait_g(b):
            pltpu.make_async_copy(table.at[sidx.at[b]], rows.at[b],
                                  sems_g[b]).wait()

        def scatter(b):
            pltpu.sync_copy(rows.at[b], agg_sh.at[didx.at[b]], add=True)

        issue_a(0, 0)
        issue_a(1, 1)
        wait_a(0)
        issue_g(0)

        def pair(j, carry):
            k = 2 * j
            wait_g(0)
            wait_a(1)
            issue_g(1)
            scatter(0)
            issue_a(k + 2, 0)
            wait_g(1)
            wait_a(0)
            issue_g(0)
            scatter(1)
            issue_a(k + 3, 1)
            return carry

        lax.fori_loop(0, k_count // 2, pair, 0)
        wait_g(0)
        wait_a(1)
        plsc.subcore_barrier()

        row0 = c * N_PAD + s * ROWS_PER_TILE
        pltpu.sync_copy(agg_sh.at[pl.ds(s * ROWS_PER_TILE, ROWS_PER_TILE)],
                        agg_out.at[pl.ds(row0, ROWS_PER_TILE)])

    return body


def _make_sc_agg(edge_split):
    return pl.kernel(
        _make_sc_agg_body(edge_split),
        out_type=jax.ShapeDtypeStruct((2 * N_PAD, 128), jnp.float32),
        mesh=plsc.VectorSubcoreMesh(core_axis_name="c", subcore_axis_name="s"),
        scratch_types=(
            pltpu.VMEM((2, CHUNK), jnp.int32),
            pltpu.VMEM((2, CHUNK), jnp.int32),
            pltpu.VMEM((2, CHUNK, 128), jnp.float32),
            pltpu.VMEM_SHARED((N_PAD, 128), jnp.float32),
            pltpu.SemaphoreType.DMA,
            pltpu.SemaphoreType.DMA,
            pltpu.SemaphoreType.DMA,
            pltpu.SemaphoreType.DMA,
        ),
    )


_SC_AGG_L1 = _make_sc_agg(True)
_SC_AGG_H = _make_sc_agg(False)


def _sc_deg_body(dstv, ones_in, zblk, deg_out,
                 didx, ones_v, deg_sh, sa0, sa1):
    c = lax.axis_index("c")
    s = lax.axis_index("s")

    pltpu.sync_copy(zblk, deg_sh.at[pl.ds(s * ROWS_PER_TILE, ROWS_PER_TILE)])
    pltpu.sync_copy(ones_in, ones_v)
    plsc.subcore_barrier()

    chunk0 = c * (N_CHUNKS_SC // 2) + s
    sems = (sa0, sa1)

    def issue_a(k, b):
        off = (chunk0 + k * N_SUBCORES) * CHUNK
        pltpu.async_copy(dstv.at[pl.ds(off, CHUNK)], didx.at[b], sems[b])

    def wait_a(b):
        pltpu.make_async_copy(dstv.at[pl.ds(0, CHUNK)], didx.at[b],
                              sems[b]).wait()

    issue_a(0, 0)
    issue_a(1, 1)

    def pair(j, carry):
        k = 2 * j
        wait_a(0)
        pltpu.sync_copy(ones_v, deg_sh.at[didx.at[0]], add=True)
        issue_a(k + 2, 0)
        wait_a(1)
        pltpu.sync_copy(ones_v, deg_sh.at[didx.at[1]], add=True)
        issue_a(k + 3, 1)
        return carry

    lax.fori_loop(0, K_HALF // 2, pair, 0)
    wait_a(0)
    wait_a(1)
    plsc.subcore_barrier()

    row0 = c * N_PAD + s * ROWS_PER_TILE
    pltpu.sync_copy(deg_sh.at[pl.ds(s * ROWS_PER_TILE, ROWS_PER_TILE)],
                    deg_out.at[pl.ds(row0, ROWS_PER_TILE)])


_SC_DEG = pl.kernel(
    _sc_deg_body,
    out_type=jax.ShapeDtypeStruct((2 * N_PAD, 128), jnp.float32),
    mesh=plsc.VectorSubcoreMesh(core_axis_name="c", subcore_axis_name="s"),
    scratch_types=(
        pltpu.VMEM((2, CHUNK), jnp.int32),
        pltpu.VMEM((CHUNK, 128), jnp.float32),
        pltpu.VMEM_SHARED((N_PAD, 128), jnp.float32),
        pltpu.SemaphoreType.DMA,
        pltpu.SemaphoreType.DMA,
    ),
)


def _tc_l1_body(h, p0, p1, d0, d1, ws, wn, b, out, deg_out):
    degsum = d0[...] + d1[...]
    inv = 1.0 / jnp.maximum(degsum, 1.0)
    dot = functools.partial(jnp.dot, preferred_element_type=jnp.float32,
                            precision=lax.Precision.HIGHEST)
    acc = dot(h[...], ws[...]) + dot((p0[...] + p1[...]) * inv, wn[...])
    acc += b[...]
    acc = jnp.maximum(acc, 0.0)
    out[0] = acc[:, :128]
    out[1] = acc[:, 128:]
    deg_out[...] = degsum


def _tc_layer_body(relu, split_out, h0, h1, a0, a1, deg, ws0, ws1, wn0, wn1, b,
                   out):
    inv = 1.0 / jnp.maximum(deg[...], 1.0)
    dot = functools.partial(jnp.dot, preferred_element_type=jnp.float32,
                            precision=lax.Precision.HIGHEST)
    acc = dot(h0[...], ws0[...]) + dot(h1[...], ws1[...])
    acc += dot(a0[...] * inv, wn0[...]) + dot(a1[...] * inv, wn1[...])
    acc += b[...]
    if relu:
        acc = jnp.maximum(acc, 0.0)
    if split_out:
        out[0] = acc[:, :128]
        out[1] = acc[:, 128:]
    else:
        out[...] = acc


_BM = 1000


def _make_tc_l1():
    bm = _BM
    in_specs = [
        pl.BlockSpec((bm, 128), lambda m: (m, 0)),
        pl.BlockSpec((bm, 128), lambda m: (m, 0)),
        pl.BlockSpec((bm, 128), lambda m: (m, 0)),
        pl.BlockSpec((bm, 1), lambda m: (m, 0)),
        pl.BlockSpec((bm, 1), lambda m: (m, 0)),
        pl.BlockSpec((128, 256), lambda m: (0, 0)),
        pl.BlockSpec((128, 256), lambda m: (0, 0)),
        pl.BlockSpec((1, 256), lambda m: (0, 0)),
    ]
    return pl.pallas_call(
        _tc_l1_body,
        grid=(N_NODES // bm,),
        in_specs=in_specs,
        out_specs=(pl.BlockSpec((2, bm, 128), lambda m: (0, m, 0)),
                   pl.BlockSpec((bm, 1), lambda m: (m, 0))),
        out_shape=(jax.ShapeDtypeStruct((2, N_NODES, 128), jnp.float32),
                   jax.ShapeDtypeStruct((N_NODES, 1), jnp.float32)),
    )


def _make_tc_layer(relu, split_out):
    bm = _BM
    in_specs = [
        pl.BlockSpec((bm, 128), lambda m: (m, 0)),
        pl.BlockSpec((bm, 128), lambda m: (m, 0)),
        pl.BlockSpec((bm, 128), lambda m: (m, 0)),
        pl.BlockSpec((bm, 128), lambda m: (m, 0)),
        pl.BlockSpec((bm, 1), lambda m: (m, 0)),
        pl.BlockSpec((128, 256), lambda m: (0, 0)),
        pl.BlockSpec((128, 256), lambda m: (0, 0)),
        pl.BlockSpec((128, 256), lambda m: (0, 0)),
        pl.BlockSpec((128, 256), lambda m: (0, 0)),
        pl.BlockSpec((1, 256), lambda m: (0, 0)),
    ]
    if split_out:
        out_shape = jax.ShapeDtypeStruct((2, N_NODES, 128), jnp.float32)
        out_spec = pl.BlockSpec((2, bm, 128), lambda m: (0, m, 0))
    else:
        out_shape = jax.ShapeDtypeStruct((N_NODES, 256), jnp.float32)
        out_spec = pl.BlockSpec((bm, 256), lambda m: (m, 0))
    return pl.pallas_call(
        functools.partial(_tc_layer_body, relu, split_out),
        grid=(N_NODES // bm,),
        in_specs=in_specs,
        out_specs=out_spec,
        out_shape=out_shape,
    )


_TC_L1 = _make_tc_l1()
_TC_L2 = _make_tc_layer(True, True)
_TC_L3 = _make_tc_layer(False, False)


def kernel(x, edge_index, W_self1, W_neigh1, b1, W_self2, W_neigh2, b2,
           W_self3, W_neigh3, b3):
    n = N_NODES
    src = edge_index[0].astype(jnp.int32)
    dst = edge_index[1].astype(jnp.int32)

    src_p = jnp.concatenate([src, jnp.zeros((E_BUF - N_EDGES,), jnp.int32)])
    dst_p = jnp.concatenate([
        dst,
        jnp.full((E_SCAT - N_EDGES,), TRASH, jnp.int32),
        jnp.zeros((E_BUF - E_SCAT,), jnp.int32),
    ])
    srcx = jnp.concatenate([src_p, src_p + n])

    z128 = jnp.zeros((ROWS_PER_TILE, 128), jnp.float32)
    ones128 = jnp.ones((CHUNK, 128), jnp.float32)

    degp = _SC_DEG(dst_p, ones128, z128)
    agg1 = _SC_AGG_L1(x, src_p, dst_p, z128)
    h, degc = _TC_L1(x, agg1[:n], agg1[N_PAD:N_PAD + n],
                     degp[:n, 0:1], degp[N_PAD:N_PAD + n, 0:1],
                     W_self1, W_neigh1, b1.reshape(1, -1))
    h2 = h.reshape(2 * n, 128)

    agg2 = _SC_AGG_H(h2, srcx, dst_p, z128)
    h = _TC_L2(h2[:n], h2[n:], agg2[:n], agg2[N_PAD:N_PAD + n], degc,
               W_self2[:128], W_self2[128:], W_neigh2[:128], W_neigh2[128:],
               b2.reshape(1, -1))
    h3 = h.reshape(2 * n, 128)

    agg3 = _SC_AGG_H(h3, srcx, dst_p, z128)
    out = _TC_L3(h3[:n], h3[n:], agg3[:n], agg3[N_PAD:N_PAD + n], degc,
                 W_self3[:128], W_self3[128:], W_neigh3[:128], W_neigh3[128:],
                 b3.reshape(1, -1))
    return out


# speedup vs baseline: 1.0004x; 1.0004x over previous
import functools

import jax
import jax.numpy as jnp
from jax import lax
from jax.experimental import pallas as pl
from jax.experimental.pallas import tpu as pltpu
from jax.experimental.pallas import tpu_sc as plsc

N_NODES = 10000
N_EDGES = 320000
N_PAD = 10240
ROWS_PER_TILE = 640
CHUNK = 128
N_SUBCORES = 16

N_CHUNKS_SC = 2560
N_CHUNKS_BUF = 2624
E_SCAT = N_CHUNKS_SC * CHUNK
E_BUF = N_CHUNKS_BUF * CHUNK
K_FULL = N_CHUNKS_SC // N_SUBCORES
K_HALF = N_CHUNKS_SC // (2 * N_SUBCORES)
TRASH = N_NODES


def _issue_idx(src_hbm, src_off, dst_hbm, dst_off, sidx_b, didx_b, sem):
    pltpu.async_copy(src_hbm.at[pl.ds(src_off, CHUNK)], sidx_b, sem)
    pltpu.async_copy(dst_hbm.at[pl.ds(dst_off, CHUNK)], didx_b, sem)


def _wait_idx(src_hbm, dst_hbm, sidx_b, didx_b, sem):
    pltpu.make_async_copy(src_hbm.at[pl.ds(0, CHUNK)], sidx_b, sem).wait()
    pltpu.make_async_copy(dst_hbm.at[pl.ds(0, CHUNK)], didx_b, sem).wait()


def _make_sc_agg_body(edge_split):
    def body(table, srcv, dstv, zblk, agg_out,
             sidx, didx, rows, agg_sh, sa0, sa1, sg0, sg1, *_):
        c = lax.axis_index("c")
        s = lax.axis_index("s")

        pltpu.sync_copy(zblk, agg_sh.at[pl.ds(s * ROWS_PER_TILE, ROWS_PER_TILE)])
        plsc.subcore_barrier()

        if edge_split:
            k_count = K_HALF
            chunk0 = c * (N_CHUNKS_SC // 2) + s
            src_base = 0
        else:
            k_count = K_FULL
            chunk0 = s
            src_base = c * E_BUF

        sems_a = (sa0, sa1)
        sems_g = (sg0, sg1)

        def issue_a(k, b):
            off = (chunk0 + k * N_SUBCORES) * CHUNK
            _issue_idx(srcv, src_base + off, dstv, off,
                       sidx.at[b], didx.at[b], sems_a[b])

        def wait_a(b):
            _wait_idx(srcv, dstv, sidx.at[b], didx.at[b], sems_a[b])

        def issue_g(b):
            pltpu.async_copy(table.at[sidx.at[b]], rows.at[b], sems_g[b])

        def wait_g(b):
            pltpu.make_async_copy(table.at[sidx.at[b]], rows.at[b],
                                  sems_g[b]).wait()

        def scatter(b):
            pltpu.sync_copy(rows.at[b], agg_sh.at[didx.at[b]], add=True)

        issue_a(0, 0)
        issue_a(1, 1)
        wait_a(0)
        issue_g(0)

        def pair(j, carry):
            k = 2 * j
            wait_g(0)
            wait_a(1)
            issue_g(1)
            scatter(0)
            issue_a(k + 2, 0)
            wait_g(1)
            wait_a(0)
            issue_g(0)
            scatter(1)
            issue_a(k + 3, 1)
            return carry

        lax.fori_loop(0, k_count // 2, pair, 0)
        wait_g(0)
        wait_a(1)
        plsc.subcore_barrier()

        row0 = c * N_PAD + s * ROWS_PER_TILE
        pltpu.sync_copy(agg_sh.at[pl.ds(s * ROWS_PER_TILE, ROWS_PER_TILE)],
                        agg_out.at[pl.ds(row0, ROWS_PER_TILE)])

    return body


def _make_sc_agg(edge_split):
    return pl.kernel(
        _make_sc_agg_body(edge_split),
        out_type=jax.ShapeDtypeStruct((2 * N_PAD, 128), jnp.float32),
        mesh=plsc.VectorSubcoreMesh(core_axis_name="c", subcore_axis_name="s"),
        scratch_types=(
            pltpu.VMEM((2, CHUNK), jnp.int32),
            pltpu.VMEM((2, CHUNK), jnp.int32),
            pltpu.VMEM((2, CHUNK, 128), jnp.float32),
            pltpu.VMEM_SHARED((N_PAD, 128), jnp.float32),
            pltpu.SemaphoreType.DMA,
            pltpu.SemaphoreType.DMA,
            pltpu.SemaphoreType.DMA,
            pltpu.SemaphoreType.DMA,
        ),
    )


_SC_AGG_L1 = _make_sc_agg(True)
_SC_AGG_H = _make_sc_agg(False)


def _sc_deg_body(dstv, ones_in, zblk, deg_out,
                 didx, ones_v, deg_sh, sa0, sa1):
    c = lax.axis_index("c")
    s = lax.axis_index("s")

    pltpu.sync_copy(zblk, deg_sh.at[pl.ds(s * ROWS_PER_TILE, ROWS_PER_TILE)])
    pltpu.sync_copy(ones_in, ones_v)
    plsc.subcore_barrier()

    chunk0 = c * (N_CHUNKS_SC // 2) + s
    sems = (sa0, sa1)

    def issue_a(k, b):
        off = (chunk0 + k * N_SUBCORES) * CHUNK
        pltpu.async_copy(dstv.at[pl.ds(off, CHUNK)], didx.at[b], sems[b])

    def wait_a(b):
        pltpu.make_async_copy(dstv.at[pl.ds(0, CHUNK)], didx.at[b],
                              sems[b]).wait()

    issue_a(0, 0)
    issue_a(1, 1)

    def pair(j, carry):
        k = 2 * j
        wait_a(0)
        pltpu.sync_copy(ones_v, deg_sh.at[didx.at[0]], add=True)
        issue_a(k + 2, 0)
        wait_a(1)
        pltpu.sync_copy(ones_v, deg_sh.at[didx.at[1]], add=True)
        issue_a(k + 3, 1)
        return carry

    lax.fori_loop(0, K_HALF // 2, pair, 0)
    wait_a(0)
    wait_a(1)
    plsc.subcore_barrier()

    row0 = c * N_PAD + s * ROWS_PER_TILE
    pltpu.sync_copy(deg_sh.at[pl.ds(s * ROWS_PER_TILE, ROWS_PER_TILE)],
                    deg_out.at[pl.ds(row0, ROWS_PER_TILE)])


_SC_DEG = pl.kernel(
    _sc_deg_body,
    out_type=jax.ShapeDtypeStruct((2 * N_PAD, 128), jnp.float32),
    mesh=plsc.VectorSubcoreMesh(core_axis_name="c", subcore_axis_name="s"),
    scratch_types=(
        pltpu.VMEM((2, CHUNK), jnp.int32),
        pltpu.VMEM((CHUNK, 128), jnp.float32),
        pltpu.VMEM_SHARED((N_PAD, 128), jnp.float32),
        pltpu.SemaphoreType.DMA,
        pltpu.SemaphoreType.DMA,
    ),
)


def _tc_l1_body(h, p0, p1, d0, d1, ws, wn, b, out, deg_out):
    degsum = d0[...] + d1[...]
    inv = 1.0 / jnp.maximum(degsum, 1.0)
    dot = functools.partial(jnp.dot, preferred_element_type=jnp.float32,
                            precision=lax.Precision.HIGHEST)
    acc = dot(h[...], ws[...]) + dot((p0[...] + p1[...]) * inv, wn[...])
    acc += b[...]
    acc = jnp.maximum(acc, 0.0)
    out[0] = acc[:, :128]
    out[1] = acc[:, 128:]
    deg_out[...] = degsum


def _tc_layer_body(relu, split_out, h0, h1, a0, a1, deg, ws0, ws1, wn0, wn1, b,
                   out):
    inv = 1.0 / jnp.maximum(deg[...], 1.0)
    dot = functools.partial(jnp.dot, preferred_element_type=jnp.float32,
                            precision=lax.Precision.HIGHEST)
    acc = dot(h0[...], ws0[...]) + dot(h1[...], ws1[...])
    acc += dot(a0[...] * inv, wn0[...]) + dot(a1[...] * inv, wn1[...])
    acc += b[...]
    if relu:
        acc = jnp.maximum(acc, 0.0)
    if split_out:
        out[0] = acc[:, :128]
        out[1] = acc[:, 128:]
    else:
        out[...] = acc


_BM = 1000


def _make_tc_l1():
    bm = _BM
    in_specs = [
        pl.BlockSpec((bm, 128), lambda m: (m, 0)),
        pl.BlockSpec((bm, 128), lambda m: (m, 0)),
        pl.BlockSpec((bm, 128), lambda m: (m, 0)),
        pl.BlockSpec((bm, 1), lambda m: (m, 0)),
        pl.BlockSpec((bm, 1), lambda m: (m, 0)),
        pl.BlockSpec((128, 256), lambda m: (0, 0)),
        pl.BlockSpec((128, 256), lambda m: (0, 0)),
        pl.BlockSpec((1, 256), lambda m: (0, 0)),
    ]
    return pl.pallas_call(
        _tc_l1_body,
        grid=(N_NODES // bm,),
        in_specs=in_specs,
        out_specs=(pl.BlockSpec((2, bm, 128), lambda m: (0, m, 0)),
                   pl.BlockSpec((bm, 1), lambda m: (m, 0))),
        out_shape=(jax.ShapeDtypeStruct((2, N_NODES, 128), jnp.float32),
                   jax.ShapeDtypeStruct((N_NODES, 1), jnp.float32)),
    )


def _make_tc_layer(relu, split_out):
    bm = _BM
    in_specs = [
        pl.BlockSpec((bm, 128), lambda m: (m, 0)),
        pl.BlockSpec((bm, 128), lambda m: (m, 0)),
        pl.BlockSpec((bm, 128), lambda m: (m, 0)),
        pl.BlockSpec((bm, 128), lambda m: (m, 0)),
        pl.BlockSpec((bm, 1), lambda m: (m, 0)),
        pl.BlockSpec((128, 256), lambda m: (0, 0)),
        pl.BlockSpec((128, 256), lambda m: (0, 0)),
        pl.BlockSpec((128, 256), lambda m: (0, 0)),
        pl.BlockSpec((128, 256), lambda m: (0, 0)),
        pl.BlockSpec((1, 256), lambda m: (0, 0)),
    ]
    if split_out:
        out_shape = jax.ShapeDtypeStruct((2, N_NODES, 128), jnp.float32)
        out_spec = pl.BlockSpec((2, bm, 128), lambda m: (0, m, 0))
    else:
        out_shape = jax.ShapeDtypeStruct((N_NODES, 256), jnp.float32)
        out_spec = pl.BlockSpec((bm, 256), lambda m: (m, 0))
    return pl.pallas_call(
        functools.partial(_tc_layer_body, relu, split_out),
        grid=(N_NODES // bm,),
        in_specs=in_specs,
        out_specs=out_spec,
        out_shape=out_shape,
    )


_TC_L1 = _make_tc_l1()
_TC_L2 = _make_tc_layer(True, True)
_TC_L3 = _make_tc_layer(False, False)


def kernel(x, edge_index, W_self1, W_neigh1, b1, W_self2, W_neigh2, b2,
           W_self3, W_neigh3, b3):
    n = N_NODES
    src = edge_index[0].astype(jnp.int32)
    dst = edge_index[1].astype(jnp.int32)

    src_p = jnp.concatenate([src, jnp.zeros((E_BUF - N_EDGES,), jnp.int32)])
    trash = TRASH + jnp.arange(E_SCAT - N_EDGES, dtype=jnp.int32) % (N_PAD - N_NODES)
    dst_p = jnp.concatenate([
        dst,
        trash,
        jnp.zeros((E_BUF - E_SCAT,), jnp.int32),
    ])
    srcx = jnp.concatenate([src_p, src_p + n])

    z128 = jnp.zeros((ROWS_PER_TILE, 128), jnp.float32)
    ones128 = jnp.ones((CHUNK, 128), jnp.float32)

    degp = _SC_DEG(dst_p, ones128, z128)
    agg1 = _SC_AGG_L1(x, src_p, dst_p, z128)
    h, degc = _TC_L1(x, agg1[:n], agg1[N_PAD:N_PAD + n],
                     degp[:n, 0:1], degp[N_PAD:N_PAD + n, 0:1],
                     W_self1, W_neigh1, b1.reshape(1, -1))
    h2 = h.reshape(2 * n, 128)

    agg2 = _SC_AGG_H(h2, srcx, dst_p, z128)
    h = _TC_L2(h2[:n], h2[n:], agg2[:n], agg2[N_PAD:N_PAD + n], degc,
               W_self2[:128], W_self2[128:], W_neigh2[:128], W_neigh2[128:],
               b2.reshape(1, -1))
    h3 = h.reshape(2 * n, 128)

    agg3 = _SC_AGG_H(h3, srcx, dst_p, z128)
    out = _TC_L3(h3[:n], h3[n:], agg3[:n], agg3[N_PAD:N_PAD + n], degc,
                 W_self3[:128], W_self3[128:], W_neigh3[:128], W_neigh3[128:],
                 b3.reshape(1, -1))
    return out


# speedup vs baseline: 2.2469x; 2.2461x over previous
import functools

import jax
import jax.numpy as jnp
from jax import lax
from jax.experimental import pallas as pl
from jax.experimental.pallas import tpu as pltpu
from jax.experimental.pallas import tpu_sc as plsc

N_NODES = 10000
N_EDGES = 320000
N_PAD = 10240
ROWS_PER_TILE = 640
CHUNK = 128
N_SUBCORES = 16

N_CHUNKS_SC = 2560
N_CHUNKS_BUF = 2624
E_SCAT = N_CHUNKS_SC * CHUNK
E_BUF = N_CHUNKS_BUF * CHUNK
K_FULL = N_CHUNKS_SC // N_SUBCORES
K_HALF = N_CHUNKS_SC // (2 * N_SUBCORES)
TRASH = N_NODES


def _issue_idx(src_hbm, src_off, dst_hbm, dst_off, sidx_b, didx_b, sem):
    pltpu.async_copy(src_hbm.at[pl.ds(src_off, CHUNK)], sidx_b, sem)
    pltpu.async_copy(dst_hbm.at[pl.ds(dst_off, CHUNK)], didx_b, sem)


def _wait_idx(src_hbm, dst_hbm, sidx_b, didx_b, sem):
    pltpu.make_async_copy(src_hbm.at[pl.ds(0, CHUNK)], sidx_b, sem).wait()
    pltpu.make_async_copy(dst_hbm.at[pl.ds(0, CHUNK)], didx_b, sem).wait()


def _make_sc_agg_body(edge_split):
    def body(table, srcv, dstv, zblk, agg_out,
             sidx, didx, rows, agg_sh, sa0, sa1, sg0, sg1, *_):
        c = lax.axis_index("c")
        s = lax.axis_index("s")

        pltpu.sync_copy(zblk, agg_sh.at[pl.ds(s * ROWS_PER_TILE, ROWS_PER_TILE)])
        plsc.subcore_barrier()

        if edge_split:
            k_count = K_HALF
            chunk0 = c * (N_CHUNKS_SC // 2) + s
            src_base = 0
        else:
            k_count = K_FULL
            chunk0 = s
            src_base = c * E_BUF

        sems_a = (sa0, sa1)
        sems_g = (sg0, sg1)

        def issue_a(k, b):
            off = (chunk0 + k * N_SUBCORES) * CHUNK
            _issue_idx(srcv, src_base + off, dstv, off,
                       sidx.at[b], didx.at[b], sems_a[b])

        def wait_a(b):
            _wait_idx(srcv, dstv, sidx.at[b], didx.at[b], sems_a[b])

        def issue_g(b):
            pltpu.async_copy(table.at[sidx.at[b]], rows.at[b], sems_g[b])

        def wait_g(b):
            pltpu.make_async_copy(table.at[sidx.at[b]], rows.at[b],
                                  sems_g[b]).wait()

        def scatter(b):
            pltpu.sync_copy(rows.at[b], agg_sh.at[didx.at[b]], add=True)

        issue_a(0, 0)
        issue_a(1, 1)
        wait_a(0)
        issue_g(0)

        def pair(j, carry):
            k = 2 * j
            wait_g(0)
            wait_a(1)
            issue_g(1)
            scatter(0)
            issue_a(k + 2, 0)
            wait_g(1)
            wait_a(0)
            issue_g(0)
            scatter(1)
            issue_a(k + 3, 1)
            return carry

        lax.fori_loop(0, k_count // 2, pair, 0)
        wait_g(0)
        wait_a(1)
        plsc.subcore_barrier()

        row0 = c * N_PAD + s * ROWS_PER_TILE
        pltpu.sync_copy(agg_sh.at[pl.ds(s * ROWS_PER_TILE, ROWS_PER_TILE)],
                        agg_out.at[pl.ds(row0, ROWS_PER_TILE)])

    return body


def _make_sc_agg(edge_split):
    return pl.kernel(
        _make_sc_agg_body(edge_split),
        out_type=jax.ShapeDtypeStruct((2 * N_PAD, 128), jnp.float32),
        mesh=plsc.VectorSubcoreMesh(core_axis_name="c", subcore_axis_name="s"),
        scratch_types=(
            pltpu.VMEM((2, CHUNK), jnp.int32),
            pltpu.VMEM((2, CHUNK), jnp.int32),
            pltpu.VMEM((2, CHUNK, 128), jnp.float32),
            pltpu.VMEM_SHARED((N_PAD, 128), jnp.float32),
            pltpu.SemaphoreType.DMA,
            pltpu.SemaphoreType.DMA,
            pltpu.SemaphoreType.DMA,
            pltpu.SemaphoreType.DMA,
        ),
    )


_SC_AGG_L1 = _make_sc_agg(True)
_SC_AGG_H = _make_sc_agg(False)


def _sc_deg_body(dstv, ones_in, zblk, deg_out,
                 didx, ones_v, deg_sh, sa0, sa1):
    c = lax.axis_index("c")
    s = lax.axis_index("s")

    pltpu.sync_copy(zblk, deg_sh.at[pl.ds(s * ROWS_PER_TILE, ROWS_PER_TILE)])
    pltpu.sync_copy(ones_in, ones_v)
    plsc.subcore_barrier()

    chunk0 = c * (N_CHUNKS_SC // 2) + s
    sems = (sa0, sa1)

    def issue_a(k, b):
        off = (chunk0 + k * N_SUBCORES) * CHUNK
        pltpu.async_copy(dstv.at[pl.ds(off, CHUNK)], didx.at[b], sems[b])

    def wait_a(b):
        pltpu.make_async_copy(dstv.at[pl.ds(0, CHUNK)], didx.at[b],
                              sems[b]).wait()

    issue_a(0, 0)
    issue_a(1, 1)

    def pair(j, carry):
        k = 2 * j
        wait_a(0)
        pltpu.sync_copy(ones_v, deg_sh.at[didx.at[0]], add=True)
        issue_a(k + 2, 0)
        wait_a(1)
        pltpu.sync_copy(ones_v, deg_sh.at[didx.at[1]], add=True)
        issue_a(k + 3, 1)
        return carry

    lax.fori_loop(0, K_HALF // 2, pair, 0)
    wait_a(0)
    wait_a(1)
    plsc.subcore_barrier()

    row0 = c * N_PAD + s * ROWS_PER_TILE
    pltpu.sync_copy(deg_sh.at[pl.ds(s * ROWS_PER_TILE, ROWS_PER_TILE)],
                    deg_out.at[pl.ds(row0, ROWS_PER_TILE)])


_SC_DEG = pl.kernel(
    _sc_deg_body,
    out_type=jax.ShapeDtypeStruct((2 * N_PAD, 128), jnp.float32),
    mesh=plsc.VectorSubcoreMesh(core_axis_name="c", subcore_axis_name="s"),
    scratch_types=(
        pltpu.VMEM((2, CHUNK), jnp.int32),
        pltpu.VMEM((CHUNK, 128), jnp.float32),
        pltpu.VMEM_SHARED((N_PAD, 128), jnp.float32),
        pltpu.SemaphoreType.DMA,
        pltpu.SemaphoreType.DMA,
    ),
)


def _tc_l1_body(h, p0, p1, d0, d1, ws, wn, b, out, deg_out):
    degsum = d0[...] + d1[...]
    inv = 1.0 / jnp.maximum(degsum, 1.0)
    dot = functools.partial(jnp.dot, preferred_element_type=jnp.float32,
                            precision=lax.Precision.HIGHEST)
    acc = dot(h[...], ws[...]) + dot((p0[...] + p1[...]) * inv, wn[...])
    acc += b[...]
    acc = jnp.maximum(acc, 0.0)
    out[0] = acc[:, :128]
    out[1] = acc[:, 128:]
    deg_out[...] = degsum


def _tc_layer_body(relu, split_out, h0, h1, a0, a1, deg, ws0, ws1, wn0, wn1, b,
                   out):
    inv = 1.0 / jnp.maximum(deg[...], 1.0)
    dot = functools.partial(jnp.dot, preferred_element_type=jnp.float32,
                            precision=lax.Precision.HIGHEST)
    acc = dot(h0[...], ws0[...]) + dot(h1[...], ws1[...])
    acc += dot(a0[...] * inv, wn0[...]) + dot(a1[...] * inv, wn1[...])
    acc += b[...]
    if relu:
        acc = jnp.maximum(acc, 0.0)
    if split_out:
        out[0] = acc[:, :128]
        out[1] = acc[:, 128:]
    else:
        out[...] = acc


_BM = 1000


def _make_tc_l1():
    bm = _BM
    in_specs = [
        pl.BlockSpec((bm, 128), lambda m: (m, 0)),
        pl.BlockSpec((bm, 128), lambda m: (m, 0)),
        pl.BlockSpec((bm, 128), lambda m: (m, 0)),
        pl.BlockSpec((bm, 1), lambda m: (m, 0)),
        pl.BlockSpec((bm, 1), lambda m: (m, 0)),
        pl.BlockSpec((128, 256), lambda m: (0, 0)),
        pl.BlockSpec((128, 256), lambda m: (0, 0)),
        pl.BlockSpec((1, 256), lambda m: (0, 0)),
    ]
    return pl.pallas_call(
        _tc_l1_body,
        grid=(N_NODES // bm,),
        in_specs=in_specs,
        out_specs=(pl.BlockSpec((2, bm, 128), lambda m: (0, m, 0)),
                   pl.BlockSpec((bm, 1), lambda m: (m, 0))),
        out_shape=(jax.ShapeDtypeStruct((2, N_NODES, 128), jnp.float32),
                   jax.ShapeDtypeStruct((N_NODES, 1), jnp.float32)),
    )


def _make_tc_layer(relu, split_out):
    bm = _BM
    in_specs = [
        pl.BlockSpec((bm, 128), lambda m: (m, 0)),
        pl.BlockSpec((bm, 128), lambda m: (m, 0)),
        pl.BlockSpec((bm, 128), lambda m: (m, 0)),
        pl.BlockSpec((bm, 128), lambda m: (m, 0)),
        pl.BlockSpec((bm, 1), lambda m: (m, 0)),
        pl.BlockSpec((128, 256), lambda m: (0, 0)),
        pl.BlockSpec((128, 256), lambda m: (0, 0)),
        pl.BlockSpec((128, 256), lambda m: (0, 0)),
        pl.BlockSpec((128, 256), lambda m: (0, 0)),
        pl.BlockSpec((1, 256), lambda m: (0, 0)),
    ]
    if split_out:
        out_shape = jax.ShapeDtypeStruct((2, N_NODES, 128), jnp.float32)
        out_spec = pl.BlockSpec((2, bm, 128), lambda m: (0, m, 0))
    else:
        out_shape = jax.ShapeDtypeStruct((N_NODES, 256), jnp.float32)
        out_spec = pl.BlockSpec((bm, 256), lambda m: (m, 0))
    return pl.pallas_call(
        functools.partial(_tc_layer_body, relu, split_out),
        grid=(N_NODES // bm,),
        in_specs=in_specs,
        out_specs=out_spec,
        out_shape=out_shape,
    )


_TC_L1 = _make_tc_l1()
_TC_L2 = _make_tc_layer(True, True)
_TC_L3 = _make_tc_layer(False, False)


def kernel(x, edge_index, W_self1, W_neigh1, b1, W_self2, W_neigh2, b2,
           W_self3, W_neigh3, b3):
    n = N_NODES
    src = edge_index[0].astype(jnp.int32)
    dst = edge_index[1].astype(jnp.int32)

    pad_iota = jnp.arange(E_BUF - N_EDGES, dtype=jnp.int32)
    src_p = jnp.concatenate([src, pad_iota % N_NODES])
    dst_p = jnp.concatenate([
        dst,
        TRASH + pad_iota[:E_SCAT - N_EDGES] % (N_PAD - N_NODES),
        jnp.zeros((E_BUF - E_SCAT,), jnp.int32),
    ])
    srcx = jnp.concatenate([src_p, src_p + n])

    z128 = jnp.zeros((ROWS_PER_TILE, 128), jnp.float32)
    ones128 = jnp.ones((CHUNK, 128), jnp.float32)

    degp = _SC_DEG(dst_p, ones128, z128)
    agg1 = _SC_AGG_L1(x, src_p, dst_p, z128)
    h, degc = _TC_L1(x, agg1[:n], agg1[N_PAD:N_PAD + n],
                     degp[:n, 0:1], degp[N_PAD:N_PAD + n, 0:1],
                     W_self1, W_neigh1, b1.reshape(1, -1))
    h2 = h.reshape(2 * n, 128)

    agg2 = _SC_AGG_H(h2, srcx, dst_p, z128)
    h = _TC_L2(h2[:n], h2[n:], agg2[:n], agg2[N_PAD:N_PAD + n], degc,
               W_self2[:128], W_self2[128:], W_neigh2[:128], W_neigh2[128:],
               b2.reshape(1, -1))
    h3 = h.reshape(2 * n, 128)

    agg3 = _SC_AGG_H(h3, srcx, dst_p, z128)
    out = _TC_L3(h3[:n], h3[n:], agg3[:n], agg3[N_PAD:N_PAD + n], degc,
                 W_self3[:128], W_self3[128:], W_neigh3[:128], W_neigh3[128:],
                 b3.reshape(1, -1))
    return out


# speedup vs baseline: 2.3086x; 1.0275x over previous
import functools

import jax
import jax.numpy as jnp
from jax import lax
from jax.experimental import pallas as pl
from jax.experimental.pallas import tpu as pltpu
from jax.experimental.pallas import tpu_sc as plsc

N_NODES = 10000
N_EDGES = 320000
N_PAD = 10240
ROWS_PER_TILE = 640
CHUNK = 128
N_SUBCORES = 16

N_CHUNKS_SC = 2560
N_CHUNKS_BUF = 2624
E_SCAT = N_CHUNKS_SC * CHUNK
E_BUF = N_CHUNKS_BUF * CHUNK
K_FULL = N_CHUNKS_SC // N_SUBCORES
K_HALF = N_CHUNKS_SC // (2 * N_SUBCORES)
TRASH = N_NODES


def _issue_idx(src_hbm, src_off, dst_hbm, dst_off, sidx_b, didx_b, sem):
    pltpu.async_copy(src_hbm.at[pl.ds(src_off, CHUNK)], sidx_b, sem)
    pltpu.async_copy(dst_hbm.at[pl.ds(dst_off, CHUNK)], didx_b, sem)


def _wait_idx(src_hbm, dst_hbm, sidx_b, didx_b, sem):
    pltpu.make_async_copy(src_hbm.at[pl.ds(0, CHUNK)], sidx_b, sem).wait()
    pltpu.make_async_copy(dst_hbm.at[pl.ds(0, CHUNK)], didx_b, sem).wait()


def _make_sc_agg_body(edge_split):
    def body(table, srcv, dstv, zblk, agg_out,
             sidx, didx, rows, agg_sh, sa0, sa1, sg0, sg1, *_):
        c = lax.axis_index("c")
        s = lax.axis_index("s")

        pltpu.sync_copy(zblk, agg_sh.at[pl.ds(s * ROWS_PER_TILE, ROWS_PER_TILE)])
        plsc.subcore_barrier()

        if edge_split:
            k_count = K_HALF
            chunk0 = c * (N_CHUNKS_SC // 2) + s
            src_base = 0
        else:
            k_count = K_FULL
            chunk0 = s
            src_base = c * E_BUF

        sems_a = (sa0, sa1)
        sems_g = (sg0, sg1)

        def issue_a(k, b):
            off = (chunk0 + k * N_SUBCORES) * CHUNK
            _issue_idx(srcv, src_base + off, dstv, off,
                       sidx.at[b], didx.at[b], sems_a[b])

        def wait_a(b):
            _wait_idx(srcv, dstv, sidx.at[b], didx.at[b], sems_a[b])

        def issue_g(b):
            pltpu.async_copy(table.at[sidx.at[b]], rows.at[b], sems_g[b])

        def wait_g(b):
            pltpu.make_async_copy(table.at[sidx.at[b]], rows.at[b],
                                  sems_g[b]).wait()

        def scatter(b):
            pltpu.sync_copy(rows.at[b], agg_sh.at[didx.at[b]], add=True)

        issue_a(0, 0)
        issue_a(1, 1)
        wait_a(0)
        issue_g(0)

        def pair(j, carry):
            k = 2 * j
            wait_g(0)
            wait_a(1)
            issue_g(1)
            scatter(0)
            issue_a(k + 2, 0)
            wait_g(1)
            wait_a(0)
            issue_g(0)
            scatter(1)
            issue_a(k + 3, 1)
            return carry

        lax.fori_loop(0, k_count // 2, pair, 0)
        wait_g(0)
        wait_a(1)
        plsc.subcore_barrier()

        row0 = c * N_PAD + s * ROWS_PER_TILE
        pltpu.sync_copy(agg_sh.at[pl.ds(s * ROWS_PER_TILE, ROWS_PER_TILE)],
                        agg_out.at[pl.ds(row0, ROWS_PER_TILE)])

    return body


def _make_sc_agg(edge_split):
    return pl.kernel(
        _make_sc_agg_body(edge_split),
        out_type=jax.ShapeDtypeStruct((2 * N_PAD, 128), jnp.float32),
        mesh=plsc.VectorSubcoreMesh(core_axis_name="c", subcore_axis_name="s"),
        scratch_types=(
            pltpu.VMEM((2, CHUNK), jnp.int32),
            pltpu.VMEM((2, CHUNK), jnp.int32),
            pltpu.VMEM((2, CHUNK, 128), jnp.float32),
            pltpu.VMEM_SHARED((N_PAD, 128), jnp.float32),
            pltpu.SemaphoreType.DMA,
            pltpu.SemaphoreType.DMA,
            pltpu.SemaphoreType.DMA,
            pltpu.SemaphoreType.DMA,
        ),
    )


_SC_AGG_L1 = _make_sc_agg(True)
_SC_AGG_H = _make_sc_agg(False)


def _sc_deg_body(dstv, ones_in, zblk, deg_out,
                 didx, ones_v, deg_sh, sa0, sa1):
    c = lax.axis_index("c")
    s = lax.axis_index("s")

    pltpu.sync_copy(zblk, deg_sh.at[pl.ds(s * ROWS_PER_TILE, ROWS_PER_TILE)])
    pltpu.sync_copy(ones_in, ones_v)
    plsc.subcore_barrier()

    chunk0 = c * (N_CHUNKS_SC // 2) + s
    sems = (sa0, sa1)

    def issue_a(k, b):
        off = (chunk0 + k * N_SUBCORES) * CHUNK
        pltpu.async_copy(dstv.at[pl.ds(off, CHUNK)], didx.at[b], sems[b])

    def wait_a(b):
        pltpu.make_async_copy(dstv.at[pl.ds(0, CHUNK)], didx.at[b],
                              sems[b]).wait()

    issue_a(0, 0)
    issue_a(1, 1)

    def pair(j, carry):
        k = 2 * j
        wait_a(0)
        pltpu.sync_copy(ones_v, deg_sh.at[didx.at[0]], add=True)
        issue_a(k + 2, 0)
        wait_a(1)
        pltpu.sync_copy(ones_v, deg_sh.at[didx.at[1]], add=True)
        issue_a(k + 3, 1)
        return carry

    lax.fori_loop(0, K_HALF // 2, pair, 0)
    wait_a(0)
    wait_a(1)
    plsc.subcore_barrier()

    row0 = c * N_PAD + s * ROWS_PER_TILE
    pltpu.sync_copy(deg_sh.at[pl.ds(s * ROWS_PER_TILE, ROWS_PER_TILE)],
                    deg_out.at[pl.ds(row0, ROWS_PER_TILE)])


_SC_DEG = pl.kernel(
    _sc_deg_body,
    out_type=jax.ShapeDtypeStruct((2 * N_PAD, 128), jnp.float32),
    mesh=plsc.VectorSubcoreMesh(core_axis_name="c", subcore_axis_name="s"),
    scratch_types=(
        pltpu.VMEM((2, CHUNK), jnp.int32),
        pltpu.VMEM((CHUNK, 128), jnp.float32),
        pltpu.VMEM_SHARED((N_PAD, 128), jnp.float32),
        pltpu.SemaphoreType.DMA,
        pltpu.SemaphoreType.DMA,
    ),
)


_DOT = functools.partial(jnp.dot, preferred_element_type=jnp.float32,
                         precision=lax.Precision.HIGHEST)
_BM = 1000


def _tc_pre_body(two_part, *refs):
    if two_part:
        h0, h1, ws0, ws1, b, out = refs
        acc = _DOT(h0[...], ws0[...]) + _DOT(h1[...], ws1[...]) + b[...]
    else:
        h, ws, b, out = refs
        acc = _DOT(h[...], ws[...]) + b[...]
    out[...] = acc


def _make_tc_pre(two_part):
    bm = _BM
    hspec = pl.BlockSpec((bm, 128), lambda m: (m, 0))
    wspec = pl.BlockSpec((128, 256), lambda m: (0, 0))
    in_specs = [hspec] * (2 if two_part else 1) + \
               [wspec] * (2 if two_part else 1) + \
               [pl.BlockSpec((1, 256), lambda m: (0, 0))]
    return pl.pallas_call(
        functools.partial(_tc_pre_body, two_part),
        grid=(N_NODES // bm,),
        in_specs=in_specs,
        out_specs=pl.BlockSpec((bm, 256), lambda m: (m, 0)),
        out_shape=jax.ShapeDtypeStruct((N_NODES, 256), jnp.float32),
    )


def _tc_post1_body(slf, p0, p1, d0, d1, wn, out, deg_out):
    degsum = d0[...] + d1[...]
    inv = 1.0 / jnp.maximum(degsum, 1.0)
    acc = slf[...] + _DOT((p0[...] + p1[...]) * inv, wn[...])
    acc = jnp.maximum(acc, 0.0)
    out[0] = acc[:, :128]
    out[1] = acc[:, 128:]
    deg_out[...] = degsum


def _make_tc_post1():
    bm = _BM
    in_specs = [
        pl.BlockSpec((bm, 256), lambda m: (m, 0)),
        pl.BlockSpec((bm, 128), lambda m: (m, 0)),
        pl.BlockSpec((bm, 128), lambda m: (m, 0)),
        pl.BlockSpec((bm, 1), lambda m: (m, 0)),
        pl.BlockSpec((bm, 1), lambda m: (m, 0)),
        pl.BlockSpec((128, 256), lambda m: (0, 0)),
    ]
    return pl.pallas_call(
        _tc_post1_body,
        grid=(N_NODES // bm,),
        in_specs=in_specs,
        out_specs=(pl.BlockSpec((2, bm, 128), lambda m: (0, m, 0)),
                   pl.BlockSpec((bm, 1), lambda m: (m, 0))),
        out_shape=(jax.ShapeDtypeStruct((2, N_NODES, 128), jnp.float32),
                   jax.ShapeDtypeStruct((N_NODES, 1), jnp.float32)),
    )


def _tc_post_body(relu, split_out, slf, a0, a1, deg, wn0, wn1, out):
    inv = 1.0 / jnp.maximum(deg[...], 1.0)
    acc = slf[...] + _DOT(a0[...] * inv, wn0[...]) + _DOT(a1[...] * inv, wn1[...])
    if relu:
        acc = jnp.maximum(acc, 0.0)
    if split_out:
        out[0] = acc[:, :128]
        out[1] = acc[:, 128:]
    else:
        out[...] = acc


def _make_tc_post(relu, split_out):
    bm = _BM
    in_specs = [
        pl.BlockSpec((bm, 256), lambda m: (m, 0)),
        pl.BlockSpec((bm, 128), lambda m: (m, 0)),
        pl.BlockSpec((bm, 128), lambda m: (m, 0)),
        pl.BlockSpec((bm, 1), lambda m: (m, 0)),
        pl.BlockSpec((128, 256), lambda m: (0, 0)),
        pl.BlockSpec((128, 256), lambda m: (0, 0)),
    ]
    if split_out:
        out_shape = jax.ShapeDtypeStruct((2, N_NODES, 128), jnp.float32)
        out_spec = pl.BlockSpec((2, bm, 128), lambda m: (0, m, 0))
    else:
        out_shape = jax.ShapeDtypeStruct((N_NODES, 256), jnp.float32)
        out_spec = pl.BlockSpec((bm, 256), lambda m: (m, 0))
    return pl.pallas_call(
        functools.partial(_tc_post_body, relu, split_out),
        grid=(N_NODES // bm,),
        in_specs=in_specs,
        out_specs=out_spec,
        out_shape=out_shape,
    )


_TC_PRE1 = _make_tc_pre(False)
_TC_PRE = _make_tc_pre(True)
_TC_POST1 = _make_tc_post1()
_TC_POST2 = _make_tc_post(True, True)
_TC_POST3 = _make_tc_post(False, False)


def kernel(x, edge_index, W_self1, W_neigh1, b1, W_self2, W_neigh2, b2,
           W_self3, W_neigh3, b3):
    n = N_NODES
    src = edge_index[0].astype(jnp.int32)
    dst = edge_index[1].astype(jnp.int32)

    pad_iota = jnp.arange(E_BUF - N_EDGES, dtype=jnp.int32)
    src_p = jnp.concatenate([src, pad_iota % N_NODES])
    dst_p = jnp.concatenate([
        dst,
        TRASH + pad_iota[:E_SCAT - N_EDGES] % (N_PAD - N_NODES),
        jnp.zeros((E_BUF - E_SCAT,), jnp.int32),
    ])
    srcx = jnp.concatenate([src_p, src_p + n])

    z128 = jnp.zeros((ROWS_PER_TILE, 128), jnp.float32)
    ones128 = jnp.ones((CHUNK, 128), jnp.float32)

    degp = _SC_DEG(dst_p, ones128, z128)
    agg1 = _SC_AGG_L1(x, src_p, dst_p, z128)
    self1 = _TC_PRE1(x, W_self1, b1.reshape(1, -1))
    h, degc = _TC_POST1(self1, agg1[:n], agg1[N_PAD:N_PAD + n],
                        degp[:n, 0:1], degp[N_PAD:N_PAD + n, 0:1], W_neigh1)
    h2 = h.reshape(2 * n, 128)

    agg2 = _SC_AGG_H(h2, srcx, dst_p, z128)
    self2 = _TC_PRE(h2[:n], h2[n:], W_self2[:128], W_self2[128:],
                    b2.reshape(1, -1))
    h = _TC_POST2(self2, agg2[:n], agg2[N_PAD:N_PAD + n], degc,
                  W_neigh2[:128], W_neigh2[128:])
    h3 = h.reshape(2 * n, 128)

    agg3 = _SC_AGG_H(h3, srcx, dst_p, z128)
    self3 = _TC_PRE(h3[:n], h3[n:], W_self3[:128], W_self3[128:],
                    b3.reshape(1, -1))
    out = _TC_POST3(self3, agg3[:n], agg3[N_PAD:N_PAD + n], degc,
                    W_neigh3[:128], W_neigh3[128:])
    return out
